# baseline (speedup 1.0000x reference)
import functools

import jax
import jax.numpy as jnp
from jax import lax
from jax.experimental import pallas as pl
from jax.experimental.pallas import tpu as pltpu
from jax.experimental.pallas import tpu_sc as plsc

_TOKENS = 16384
_D_MODEL = 2048
_EXPERTS = 64
_BT = 512

_NW = 32
_TPW = _TOKENS // _NW
_LANES = 16
_GPW = _TPW // _LANES


def _top2_block(x_ref, w_ref, out_ref):
    logits_t = jax.lax.dot_general(
        w_ref[...], x_ref[...], (((1,), (1,)), ((), ())),
        preferred_element_type=jnp.float32,
    )
    iota_e = jax.lax.broadcasted_iota(jnp.int32, logits_t.shape, 0)
    m1 = jnp.max(logits_t, axis=0, keepdims=True)
    i1 = jnp.min(jnp.where(logits_t == m1, iota_e, _EXPERTS), axis=0, keepdims=True)
    masked = jnp.where(iota_e == i1, -jnp.inf, logits_t)
    m2 = jnp.max(masked, axis=0, keepdims=True)
    i2 = jnp.min(jnp.where(masked == m2, iota_e, _EXPERTS), axis=0, keepdims=True)
    g1 = jax.lax.bitcast_convert_type(jax.nn.sigmoid(m1 - m2), jnp.int32)
    g2 = jax.lax.bitcast_convert_type(1.0 - jax.nn.sigmoid(m1 - m2), jnp.int32)
    out_ref[...] = jnp.concatenate([g1, g2, i1, i2], axis=0)


def _top2(x, W):
    return pl.pallas_call(
        _top2_block,
        grid=(_TOKENS // _BT,),
        in_specs=[
            pl.BlockSpec((_BT, _D_MODEL), lambda i: (i, 0)),
            pl.BlockSpec((_EXPERTS, _D_MODEL), lambda i: (0, 0)),
        ],
        out_specs=pl.BlockSpec((4, _BT), lambda i: (0, i)),
        out_shape=jax.ShapeDtypeStruct((4, _TOKENS), jnp.int32),
        compiler_params=pltpu.CompilerParams(
            dimension_semantics=("arbitrary",),
        ),
    )(x, W)


def _scatter(top2_hbm, out_hbm, tbuf, obuf, sem):
    wid = lax.axis_index("s") * 2 + lax.axis_index("c")
    base = wid * _TPW
    dma_in = pltpu.async_copy(top2_hbm.at[:, pl.ds(base, _TPW)], tbuf, sem)

    zeros16 = jnp.zeros((_LANES,), jnp.float32)

    def zero_body(j, carry):
        for k in range(8):
            obuf[pl.ds(j * 8 * _LANES + k * _LANES, _LANES)] = zeros16
        return carry

    lax.fori_loop(0, _TPW * _EXPERTS // (8 * _LANES), zero_body, 0)
    dma_in.wait()

    lanes = lax.iota(jnp.int32, _LANES)

    def group_body(g, carry):
        t0 = g * _LANES
        flat = (t0 + lanes) * _EXPERTS
        plsc.store_scatter(obuf, [flat + tbuf[2, pl.ds(t0, _LANES)]],
                           plsc.bitcast(tbuf[0, pl.ds(t0, _LANES)], jnp.float32))
        plsc.store_scatter(obuf, [flat + tbuf[3, pl.ds(t0, _LANES)]],
                           plsc.bitcast(tbuf[1, pl.ds(t0, _LANES)], jnp.float32))
        return carry

    lax.fori_loop(0, _GPW, group_body, 0)
    pltpu.sync_copy(obuf, out_hbm.at[pl.ds(base * _EXPERTS, _TPW * _EXPERTS)])


@functools.cache
def _scatter_kernel():
    return pl.kernel(
        _scatter,
        mesh=plsc.VectorSubcoreMesh(core_axis_name="c", subcore_axis_name="s"),
        out_type=jax.ShapeDtypeStruct((_TOKENS * _EXPERTS,), jnp.float32),
        scratch_types=[
            pltpu.VMEM((4, _TPW), jnp.int32),
            pltpu.VMEM((_TPW * _EXPERTS,), jnp.float32),
            pltpu.SemaphoreType.DMA,
        ],
        compiler_params=pltpu.CompilerParams(needs_layout_passes=False),
    )


def kernel(x, W):
    top2 = _top2(x, W)
    gates_flat = _scatter_kernel()(top2)
    return gates_flat.reshape(_TOKENS, _EXPERTS)


# speedup vs baseline: 1.1146x; 1.1146x over previous
import functools

import jax
import jax.numpy as jnp
from jax import lax
from jax.experimental import pallas as pl
from jax.experimental.pallas import tpu as pltpu
from jax.experimental.pallas import tpu_sc as plsc

_TOKENS = 16384
_D_MODEL = 2048
_EXPERTS = 64
_BT = 1024

_NW = 32
_TPW = _TOKENS // _NW
_LANES = 16
_GPW = _TPW // _LANES


def _top2_block(x_ref, w_ref, out_ref):
    logits_t = jax.lax.dot_general(
        w_ref[...], x_ref[...], (((1,), (1,)), ((), ())),
        preferred_element_type=jnp.float32,
    )
    iota_e = jax.lax.broadcasted_iota(jnp.int32, logits_t.shape, 0)
    m1 = jnp.max(logits_t, axis=0, keepdims=True)
    i1 = jnp.min(jnp.where(logits_t == m1, iota_e, _EXPERTS), axis=0, keepdims=True)
    masked = jnp.where(iota_e == i1, -jnp.inf, logits_t)
    m2 = jnp.max(masked, axis=0, keepdims=True)
    i2 = jnp.min(jnp.where(masked == m2, iota_e, _EXPERTS), axis=0, keepdims=True)
    g1 = jax.lax.bitcast_convert_type(jax.nn.sigmoid(m1 - m2), jnp.int32)
    g2 = jax.lax.bitcast_convert_type(1.0 - jax.nn.sigmoid(m1 - m2), jnp.int32)
    out_ref[...] = jnp.concatenate([g1, g2, i1, i2], axis=0)


def _top2(x, W):
    return pl.pallas_call(
        _top2_block,
        grid=(_TOKENS // _BT,),
        in_specs=[
            pl.BlockSpec((_BT, _D_MODEL), lambda i: (i, 0)),
            pl.BlockSpec((_EXPERTS, _D_MODEL), lambda i: (0, 0)),
        ],
        out_specs=pl.BlockSpec((4, _BT), lambda i: (0, i)),
        out_shape=jax.ShapeDtypeStruct((4, _TOKENS), jnp.int32),
        compiler_params=pltpu.CompilerParams(
            dimension_semantics=("arbitrary",),
        ),
    )(x, W)


def _scatter(top2_hbm, out_hbm, tbuf, obuf, sem):
    wid = lax.axis_index("s") * 2 + lax.axis_index("c")
    base = wid * _TPW
    dma_in = pltpu.async_copy(top2_hbm.at[:, pl.ds(base, _TPW)], tbuf, sem)

    zeros16 = jnp.zeros((_LANES,), jnp.float32)

    def zero_body(j, carry):
        for k in range(8):
            obuf[pl.ds(j * 8 * _LANES + k * _LANES, _LANES)] = zeros16
        return carry

    lax.fori_loop(0, _TPW * _EXPERTS // (8 * _LANES), zero_body, 0)
    dma_in.wait()

    lanes = lax.iota(jnp.int32, _LANES)

    def group_body(g, carry):
        t0 = g * _LANES
        flat = (t0 + lanes) * _EXPERTS
        plsc.store_scatter(obuf, [flat + tbuf[2, pl.ds(t0, _LANES)]],
                           plsc.bitcast(tbuf[0, pl.ds(t0, _LANES)], jnp.float32))
        plsc.store_scatter(obuf, [flat + tbuf[3, pl.ds(t0, _LANES)]],
                           plsc.bitcast(tbuf[1, pl.ds(t0, _LANES)], jnp.float32))
        return carry

    lax.fori_loop(0, _GPW, group_body, 0)
    pltpu.sync_copy(obuf, out_hbm.at[pl.ds(base * _EXPERTS, _TPW * _EXPERTS)])


@functools.cache
def _scatter_kernel():
    return pl.kernel(
        _scatter,
        mesh=plsc.VectorSubcoreMesh(core_axis_name="c", subcore_axis_name="s"),
        out_type=jax.ShapeDtypeStruct((_TOKENS * _EXPERTS,), jnp.float32),
        scratch_types=[
            pltpu.VMEM((4, _TPW), jnp.int32),
            pltpu.VMEM((_TPW * _EXPERTS,), jnp.float32),
            pltpu.SemaphoreType.DMA,
        ],
        compiler_params=pltpu.CompilerParams(needs_layout_passes=False),
    )


def kernel(x, W):
    top2 = _top2(x, W)
    gates_flat = _scatter_kernel()(top2)
    return gates_flat.reshape(_TOKENS, _EXPERTS)


# speedup vs baseline: 1.1150x; 1.0004x over previous
import functools

import jax
import jax.numpy as jnp
from jax import lax
from jax.experimental import pallas as pl
from jax.experimental.pallas import tpu as pltpu
from jax.experimental.pallas import tpu_sc as plsc

_TOKENS = 16384
_D_MODEL = 2048
_EXPERTS = 64
_BT = 1024

_NW = 32
_TPW = _TOKENS // _NW
_LANES = 16
_GPW = _TPW // _LANES


def _top2_block(x_ref, w_ref, out_ref):
    logits_t = jax.lax.dot_general(
        w_ref[...], x_ref[...], (((1,), (1,)), ((), ())),
        preferred_element_type=jnp.float32,
    )
    iota_e = jax.lax.broadcasted_iota(jnp.int32, logits_t.shape, 0)
    m1 = jnp.max(logits_t, axis=0, keepdims=True)
    i1 = jnp.min(jnp.where(logits_t == m1, iota_e, _EXPERTS), axis=0, keepdims=True)
    masked = jnp.where(iota_e == i1, -jnp.inf, logits_t)
    m2 = jnp.max(masked, axis=0, keepdims=True)
    i2 = jnp.min(jnp.where(masked == m2, iota_e, _EXPERTS), axis=0, keepdims=True)
    g1 = jax.lax.bitcast_convert_type(jax.nn.sigmoid(m1 - m2), jnp.int32)
    g2 = jax.lax.bitcast_convert_type(1.0 - jax.nn.sigmoid(m1 - m2), jnp.int32)
    out_ref[...] = jnp.concatenate([g1, g2, i1, i2], axis=0)


def _top2(x, W):
    return pl.pallas_call(
        _top2_block,
        grid=(_TOKENS // _BT,),
        in_specs=[
            pl.BlockSpec((_BT, _D_MODEL), lambda i: (i, 0)),
            pl.BlockSpec((_EXPERTS, _D_MODEL), lambda i: (0, 0)),
        ],
        out_specs=pl.BlockSpec((4, _BT), lambda i: (0, i)),
        out_shape=jax.ShapeDtypeStruct((4, _TOKENS), jnp.int32),
        compiler_params=pltpu.CompilerParams(
            dimension_semantics=("parallel",),
        ),
    )(x, W)


def _scatter(top2_hbm, out_hbm, tbuf, obuf, sem):
    wid = lax.axis_index("s") * 2 + lax.axis_index("c")
    base = wid * _TPW
    dma_in = pltpu.async_copy(top2_hbm.at[:, pl.ds(base, _TPW)], tbuf, sem)

    zeros16 = jnp.zeros((_LANES,), jnp.float32)

    def zero_body(j, carry):
        for k in range(8):
            obuf[pl.ds(j * 8 * _LANES + k * _LANES, _LANES)] = zeros16
        return carry

    lax.fori_loop(0, _TPW * _EXPERTS // (8 * _LANES), zero_body, 0)
    dma_in.wait()

    lanes = lax.iota(jnp.int32, _LANES)

    def group_body(g, carry):
        t0 = g * _LANES
        flat = (t0 + lanes) * _EXPERTS
        plsc.store_scatter(obuf, [flat + tbuf[2, pl.ds(t0, _LANES)]],
                           plsc.bitcast(tbuf[0, pl.ds(t0, _LANES)], jnp.float32))
        plsc.store_scatter(obuf, [flat + tbuf[3, pl.ds(t0, _LANES)]],
                           plsc.bitcast(tbuf[1, pl.ds(t0, _LANES)], jnp.float32))
        return carry

    lax.fori_loop(0, _GPW, group_body, 0)
    pltpu.sync_copy(obuf, out_hbm.at[pl.ds(base * _EXPERTS, _TPW * _EXPERTS)])


@functools.cache
def _scatter_kernel():
    return pl.kernel(
        _scatter,
        mesh=plsc.VectorSubcoreMesh(core_axis_name="c", subcore_axis_name="s"),
        out_type=jax.ShapeDtypeStruct((_TOKENS * _EXPERTS,), jnp.float32),
        scratch_types=[
            pltpu.VMEM((4, _TPW), jnp.int32),
            pltpu.VMEM((_TPW * _EXPERTS,), jnp.float32),
            pltpu.SemaphoreType.DMA,
        ],
        compiler_params=pltpu.CompilerParams(needs_layout_passes=False),
    )


def kernel(x, W):
    top2 = _top2(x, W)
    gates_flat = _scatter_kernel()(top2)
    return gates_flat.reshape(_TOKENS, _EXPERTS)


# speedup vs baseline: 1.1192x; 1.0037x over previous
import functools

import jax
import jax.numpy as jnp
from jax import lax
from jax.experimental import pallas as pl
from jax.experimental.pallas import tpu as pltpu
from jax.experimental.pallas import tpu_sc as plsc

_TOKENS = 16384
_D_MODEL = 2048
_EXPERTS = 64
_BT = 1024

_NW = 32
_TPW = _TOKENS // _NW
_LANES = 16
_GPW = _TPW // _LANES


def _top2_block(x_ref, w_ref, out_ref):
    logits_t = jax.lax.dot_general(
        w_ref[...], x_ref[...], (((1,), (1,)), ((), ())),
        preferred_element_type=jnp.float32,
    )
    iota_e = jax.lax.broadcasted_iota(jnp.int32, logits_t.shape, 0)
    m1 = jnp.max(logits_t, axis=0, keepdims=True)
    i1 = jnp.min(jnp.where(logits_t == m1, iota_e, _EXPERTS), axis=0, keepdims=True)
    masked = jnp.where(iota_e == i1, -jnp.inf, logits_t)
    m2 = jnp.max(masked, axis=0, keepdims=True)
    i2 = jnp.min(jnp.where(masked == m2, iota_e, _EXPERTS), axis=0, keepdims=True)
    g1 = jax.lax.bitcast_convert_type(jax.nn.sigmoid(m1 - m2), jnp.int32)
    g2 = jax.lax.bitcast_convert_type(1.0 - jax.nn.sigmoid(m1 - m2), jnp.int32)
    out_ref[...] = jnp.concatenate([g1, g2, i1, i2], axis=0)


def _top2(x, W):
    return pl.pallas_call(
        _top2_block,
        grid=(_TOKENS // _BT,),
        in_specs=[
            pl.BlockSpec((_BT, _D_MODEL), lambda i: (i, 0)),
            pl.BlockSpec((_EXPERTS, _D_MODEL), lambda i: (0, 0)),
        ],
        out_specs=pl.BlockSpec((4, _BT), lambda i: (0, i)),
        out_shape=jax.ShapeDtypeStruct((4, _TOKENS), jnp.int32),
        compiler_params=pltpu.CompilerParams(
            dimension_semantics=("parallel",),
        ),
    )(x, W)


def _scatter(top2_hbm, out_hbm, tbuf, obuf, sem):
    wid = lax.axis_index("s") * 2 + lax.axis_index("c")
    base = wid * _TPW
    dma_in = pltpu.async_copy(top2_hbm.at[:, pl.ds(base, _TPW)], tbuf, sem)

    zeros16 = jnp.zeros((_LANES,), jnp.float32)

    def zero_body(j, carry):
        for k in range(8):
            obuf[pl.ds(j * 8 * _LANES + k * _LANES, _LANES)] = zeros16
        return carry

    lax.fori_loop(0, _TPW * _EXPERTS // (8 * _LANES), zero_body, 0)
    dma_in.wait()

    lanes = lax.iota(jnp.int32, _LANES)

    def group_body(g, carry):
        t0 = g * _LANES
        flat = (t0 + lanes) * _EXPERTS
        plsc.store_scatter(obuf, [flat + tbuf[2, pl.ds(t0, _LANES)]],
                           plsc.bitcast(tbuf[0, pl.ds(t0, _LANES)], jnp.float32))
        plsc.store_scatter(obuf, [flat + tbuf[3, pl.ds(t0, _LANES)]],
                           plsc.bitcast(tbuf[1, pl.ds(t0, _LANES)], jnp.float32))
        return carry

    half = _TPW * _EXPERTS // 2
    lax.fori_loop(0, _GPW // 2, group_body, 0)
    dma_out0 = pltpu.async_copy(
        obuf.at[pl.ds(0, half)],
        out_hbm.at[pl.ds(base * _EXPERTS, half)], sem)
    lax.fori_loop(_GPW // 2, _GPW, group_body, 0)
    dma_out0.wait()
    pltpu.sync_copy(obuf.at[pl.ds(half, half)],
                    out_hbm.at[pl.ds(base * _EXPERTS + half, half)])


@functools.cache
def _scatter_kernel():
    return pl.kernel(
        _scatter,
        mesh=plsc.VectorSubcoreMesh(core_axis_name="c", subcore_axis_name="s"),
        out_type=jax.ShapeDtypeStruct((_TOKENS * _EXPERTS,), jnp.float32),
        scratch_types=[
            pltpu.VMEM((4, _TPW), jnp.int32),
            pltpu.VMEM((_TPW * _EXPERTS,), jnp.float32),
            pltpu.SemaphoreType.DMA,
        ],
        compiler_params=pltpu.CompilerParams(needs_layout_passes=False),
    )


def kernel(x, W):
    top2 = _top2(x, W)
    gates_flat = _scatter_kernel()(top2)
    return gates_flat.reshape(_TOKENS, _EXPERTS)
